# baseline (speedup 1.0000x reference)
import jax
import jax.numpy as jnp
from jax.experimental import pallas as pl
from jax.experimental.pallas import tpu as pltpu

B, T, D_NUM = 16, 2048, 256
MASK_D = 32
LAB_DIM = 16
STRAIN_DIM = 8
D_OUT = D_NUM + 4 * MASK_D + LAB_DIM + 2 * STRAIN_DIM

TILE_R = 2048
NSTEP = (B * T) // TILE_R


def _body(lab_sref, astr_sref, tstr_sref,
          num_hbm, ax_hbm, ay_hbm, tx_hbm, ty_hbm,
          mean_ref, std_ref, lab_tab_ref, strain_tab_ref,
          out_hbm,
          num_v, ax_v, ay_v, tx_v, ty_v, out_v,
          in_sem, out_sem):
    i = pl.program_id(0)
    slot = jax.lax.rem(i, 2)
    nslot = jax.lax.rem(i + 1, 2)

    def start_in(step, sl):
        rows = pl.ds(step * TILE_R, TILE_R)
        pltpu.make_async_copy(num_hbm.at[rows, :], num_v.at[sl], in_sem.at[sl, 0]).start()
        pltpu.make_async_copy(ax_hbm.at[rows, :], ax_v.at[sl], in_sem.at[sl, 1]).start()
        pltpu.make_async_copy(ay_hbm.at[rows, :], ay_v.at[sl], in_sem.at[sl, 2]).start()
        pltpu.make_async_copy(tx_hbm.at[rows, :], tx_v.at[sl], in_sem.at[sl, 3]).start()
        pltpu.make_async_copy(ty_hbm.at[rows, :], ty_v.at[sl], in_sem.at[sl, 4]).start()

    def wait_in(step, sl):
        rows = pl.ds(step * TILE_R, TILE_R)
        pltpu.make_async_copy(num_hbm.at[rows, :], num_v.at[sl], in_sem.at[sl, 0]).wait()
        pltpu.make_async_copy(ax_hbm.at[rows, :], ax_v.at[sl], in_sem.at[sl, 1]).wait()
        pltpu.make_async_copy(ay_hbm.at[rows, :], ay_v.at[sl], in_sem.at[sl, 2]).wait()
        pltpu.make_async_copy(tx_hbm.at[rows, :], tx_v.at[sl], in_sem.at[sl, 3]).wait()
        pltpu.make_async_copy(ty_hbm.at[rows, :], ty_v.at[sl], in_sem.at[sl, 4]).wait()

    def out_copy(step, sl):
        rows = pl.ds(step * TILE_R, TILE_R)
        return pltpu.make_async_copy(out_v.at[sl], out_hbm.at[rows, :], out_sem.at[sl])

    @pl.when(i == 0)
    def _():
        start_in(0, 0)

    @pl.when(i + 1 < NSTEP)
    def _():
        start_in(i + 1, nslot)

    wait_in(i, slot)

    @pl.when(i >= 2)
    def _():
        out_copy(i - 2, slot).wait()

    normed = (num_v[slot] - mean_ref[0]) / std_ref[0]
    lab_vec = lab_tab_ref[pl.ds(lab_sref[i], 1), :]
    s1_vec = strain_tab_ref[pl.ds(astr_sref[i], 1), :]
    s2_vec = strain_tab_ref[pl.ds(tstr_sref[i], 1), :]
    out_v[slot] = jnp.concatenate(
        [
            normed,
            ax_v[slot], ay_v[slot], tx_v[slot], ty_v[slot],
            jnp.broadcast_to(lab_vec, (TILE_R, LAB_DIM)),
            jnp.broadcast_to(s1_vec, (TILE_R, STRAIN_DIM)),
            jnp.broadcast_to(s2_vec, (TILE_R, STRAIN_DIM)),
        ],
        axis=-1,
    )

    out_copy(i, slot).start()

    @pl.when(i == NSTEP - 1)
    def _():
        out_copy(i - 1, nslot).wait()
        out_copy(i, slot).wait()


def kernel(numeric_feats, agent_x_mask, agent_y_mask, target_x_mask,
           target_y_mask, lab_idx, agent_strain_idx, target_strain_idx,
           mean, std, emb_lab, emb_strain):
    lab_idx = lab_idx.astype(jnp.int32)
    agent_strain_idx = agent_strain_idx.astype(jnp.int32)
    target_strain_idx = target_strain_idx.astype(jnp.int32)
    mean2 = mean.reshape(1, D_NUM)
    std2 = std.reshape(1, D_NUM)
    n_rows = B * T
    num2 = numeric_feats.reshape(n_rows, D_NUM)
    ax2 = agent_x_mask.reshape(n_rows, MASK_D)
    ay2 = agent_y_mask.reshape(n_rows, MASK_D)
    tx2 = target_x_mask.reshape(n_rows, MASK_D)
    ty2 = target_y_mask.reshape(n_rows, MASK_D)

    grid_spec = pltpu.PrefetchScalarGridSpec(
        num_scalar_prefetch=3,
        grid=(NSTEP,),
        in_specs=[
            pl.BlockSpec(memory_space=pl.ANY),
            pl.BlockSpec(memory_space=pl.ANY),
            pl.BlockSpec(memory_space=pl.ANY),
            pl.BlockSpec(memory_space=pl.ANY),
            pl.BlockSpec(memory_space=pl.ANY),
            pl.BlockSpec((1, D_NUM), lambda i, *_: (0, 0)),
            pl.BlockSpec((1, D_NUM), lambda i, *_: (0, 0)),
            pl.BlockSpec(emb_lab.shape, lambda i, *_: (0, 0)),
            pl.BlockSpec(emb_strain.shape, lambda i, *_: (0, 0)),
        ],
        out_specs=pl.BlockSpec(memory_space=pl.ANY),
        scratch_shapes=[
            pltpu.VMEM((2, TILE_R, D_NUM), jnp.float32),
            pltpu.VMEM((2, TILE_R, MASK_D), jnp.float32),
            pltpu.VMEM((2, TILE_R, MASK_D), jnp.float32),
            pltpu.VMEM((2, TILE_R, MASK_D), jnp.float32),
            pltpu.VMEM((2, TILE_R, MASK_D), jnp.float32),
            pltpu.VMEM((2, TILE_R, D_OUT), jnp.float32),
            pltpu.SemaphoreType.DMA((2, 5)),
            pltpu.SemaphoreType.DMA((2,)),
        ],
    )

    out = pl.pallas_call(
        _body,
        grid_spec=grid_spec,
        out_shape=jax.ShapeDtypeStruct((n_rows, D_OUT), jnp.float32),
    )(lab_idx, agent_strain_idx, target_strain_idx,
      num2, ax2, ay2, tx2, ty2, mean2, std2, emb_lab, emb_strain)
    return out.reshape(B, T, D_OUT)
